# Initial kernel scaffold; baseline (speedup 1.0000x reference)
import jax
import jax.numpy as jnp
from jax.experimental import pallas as pl


def kernel(anchor_points_lvl0, anchor_points_lvl1, anchor_points_lvl2, anchor_points_lvl3, anchor_points_lvl4, gt_bboxes, mean, sigma, labels, inside_gt_bbox_mask):
    raise NotImplementedError("write your pallas kernel here")


# speedup vs baseline: 2.1631x; 2.1631x over previous
import numpy as np
import jax
import jax.numpy as jnp
from jax.experimental import pallas as pl

_STRIDES = (8, 16, 32, 64, 128)
_LEVEL_SIZES = (4096, 1024, 256, 64, 16)
_N = sum(_LEVEL_SIZES)
_G = 200
_C = 80
_K = 9

_INV_STRIDE = np.repeat(
    np.array([1.0 / s for s in _STRIDES], dtype=np.float32),
    np.array(_LEVEL_SIZES),
).reshape(_N, 1)


_GP = 256
_GB = 128


def _body(pts_ref, invs_ref, gtb_ref, mean_ref, sigma_ref, lab_ref, mask_ref,
          w_out_ref, m_out_ref):
    px = pts_ref[:, 0:1]
    py = pts_ref[:, 1:2]
    invs = invs_ref[...]

    cx = (gtb_ref[0:1, :] + gtb_ref[2:3, :]) * 0.5
    cy = (gtb_ref[1:2, :] + gtb_ref[3:4, :]) * 0.5

    lab = lab_ref[0:1, :]
    cls = jax.lax.broadcasted_iota(jnp.int32, (_C, _GB), 0)
    sel = (cls == lab).astype(jnp.float32)
    mu_x = jnp.sum(sel * mean_ref[:, 0:1], axis=0, keepdims=True)
    mu_y = jnp.sum(sel * mean_ref[:, 1:2], axis=0, keepdims=True)
    sg_x = jnp.sum(sel * sigma_ref[:, 0:1], axis=0, keepdims=True)
    sg_y = jnp.sum(sel * sigma_ref[:, 1:2], axis=0, keepdims=True)
    kx = 0.5 / (sg_x * sg_x)
    ky = 0.5 / (sg_y * sg_y)

    dx = (px - cx) * invs - mu_x
    dy = (py - cy) * invs - mu_y
    w = jnp.exp(-(dx * dx * kx + dy * dy * ky))

    mf = jnp.where(mask_ref[...], 1.0, 0.0)
    cnt = jnp.sum(mf, axis=0, keepdims=True)
    no_in_f = jnp.where(cnt == 0.0, 1.0, 0.0)

    rows = jax.lax.broadcasted_iota(jnp.int32, (_N, _GB), 0)
    wk = w
    keep_f = jnp.zeros((_N, _GB), jnp.float32)
    for _ in range(_K):
        mx = jnp.max(wk, axis=0, keepdims=True)
        cand = jnp.where(wk == mx, rows, _N)
        idx = jnp.min(cand, axis=0, keepdims=True)
        s_f = jnp.where(rows == idx, 1.0, 0.0)
        keep_f = keep_f + s_f
        wk = jnp.where(s_f > 0.0, -1.0, wk)

    m_out_f = mf + no_in_f * keep_f
    m_out_ref[...] = m_out_f > 0.5
    w_out_ref[...] = w * m_out_f


def kernel(anchor_points_lvl0, anchor_points_lvl1, anchor_points_lvl2,
           anchor_points_lvl3, anchor_points_lvl4, gt_bboxes, mean, sigma,
           labels, inside_gt_bbox_mask):
    pts = jnp.concatenate(
        [anchor_points_lvl0, anchor_points_lvl1, anchor_points_lvl2,
         anchor_points_lvl3, anchor_points_lvl4], axis=0)
    invs = jnp.asarray(_INV_STRIDE)
    pad = _GP - _G
    gtb_t = jnp.pad(gt_bboxes.T, ((0, 0), (0, pad)))
    lab2d = jnp.pad(labels.astype(jnp.int32).reshape(1, _G), ((0, 0), (0, pad)))
    mask_p = jnp.pad(inside_gt_bbox_mask, ((0, 0), (0, pad)))

    grid = (_GP // _GB,)
    w, m = pl.pallas_call(
        _body,
        grid=grid,
        in_specs=[
            pl.BlockSpec((_N, 2), lambda j: (0, 0)),
            pl.BlockSpec((_N, 1), lambda j: (0, 0)),
            pl.BlockSpec((4, _GB), lambda j: (0, j)),
            pl.BlockSpec((_C, 2), lambda j: (0, 0)),
            pl.BlockSpec((_C, 2), lambda j: (0, 0)),
            pl.BlockSpec((1, _GB), lambda j: (0, j)),
            pl.BlockSpec((_N, _GB), lambda j: (0, j)),
        ],
        out_specs=(
            pl.BlockSpec((_N, _GB), lambda j: (0, j)),
            pl.BlockSpec((_N, _GB), lambda j: (0, j)),
        ),
        out_shape=(
            jax.ShapeDtypeStruct((_N, _GP), jnp.float32),
            jax.ShapeDtypeStruct((_N, _GP), jnp.bool_),
        ),
    )(pts, invs, gtb_t, mean, sigma, lab2d, mask_p)
    return (w[:, :_G], m[:, :_G])
